# baseline (speedup 1.0000x reference)
import jax
import jax.numpy as jnp
from jax.experimental import pallas as pl
from jax.experimental.pallas import tpu as pltpu


def _expmap_body(x_ref, o_ref):
    x = x_ref[...]
    n2 = jnp.sum(x * x, axis=1, keepdims=True) * 0.01
    n2 = jnp.maximum(n2, 1e-14)
    r = jax.lax.rsqrt(n2)
    n = n2 * r
    t = jnp.tanh(n)
    o_ref[...] = x * (0.1 * (t * r))


def kernel(x):
    b, s, d = x.shape
    xt = jnp.transpose(x, (1, 2, 0))
    out_t = pl.pallas_call(
        _expmap_body,
        grid=(s,),
        in_specs=[pl.BlockSpec((1, d, b), lambda i: (i, 0, 0))],
        out_specs=pl.BlockSpec((1, d, b), lambda i: (i, 0, 0)),
        out_shape=jax.ShapeDtypeStruct((s, d, b), jnp.float32),
        compiler_params=pltpu.CompilerParams(
            dimension_semantics=("arbitrary",),
        ),
    )(xt)
    return jnp.transpose(out_t, (2, 0, 1))


# speedup vs baseline: 1.0005x; 1.0005x over previous
import jax
import jax.numpy as jnp
from jax.experimental import pallas as pl
from jax.experimental.pallas import tpu as pltpu


def _expmap_body(x_ref, o_ref):
    x = x_ref[...]
    n2 = jnp.sum(x * x, axis=1, keepdims=True) * 0.01
    n2 = jnp.maximum(n2, 1e-14)
    r = jax.lax.rsqrt(n2)
    n = n2 * r
    t = jnp.tanh(n)
    o_ref[...] = x * (0.1 * (t * r))


def kernel(x):
    b, s, d = x.shape
    xt = jnp.transpose(x, (1, 2, 0))
    out_t = pl.pallas_call(
        _expmap_body,
        grid=(s // 2,),
        in_specs=[pl.BlockSpec((2, d, b), lambda i: (i, 0, 0))],
        out_specs=pl.BlockSpec((2, d, b), lambda i: (i, 0, 0)),
        out_shape=jax.ShapeDtypeStruct((s, d, b), jnp.float32),
        compiler_params=pltpu.CompilerParams(
            dimension_semantics=("arbitrary",),
        ),
    )(xt)
    return jnp.transpose(out_t, (2, 0, 1))
